# baseline (speedup 1.0000x reference)
import functools

import jax
import jax.numpy as jnp
from jax import lax
from jax.experimental import pallas as pl
from jax.experimental.pallas import tpu as pltpu
from jax.experimental.pallas import tpu_sc as plsc

_VOCAB = 1000000
_EMB = 16
_B = 16384
_L = 200

_NC = 2
_NS = 16
_NW = _NC * _NS
_SPW = _B // _NW
_GRP = 16
_NGRP = _SPW // _GRP
_CHUNK = _GRP * _L
_GPC = 2
_VCHUNK = _CHUNK * _GPC
_NCHK = _NGRP // _GPC


def _scores_body(t_ref, w_ref, b_ref, o_ref):
    d = lax.dot_general(
        w_ref[:], t_ref[:],
        dimension_numbers=(((1,), (0,)), ((), ())),
        preferred_element_type=jnp.float32,
    )
    o_ref[:] = d.reshape(o_ref.shape) + b_ref[0]


_SBLK = 32768
_VOCAB_PAD = 1015808


def _vocab_scores(table, W, b):
    tT = table.T
    return pl.pallas_call(
        _scores_body,
        grid=(_VOCAB_PAD // _SBLK,),
        in_specs=[
            pl.BlockSpec((_EMB, _SBLK), lambda i: (0, i)),
            pl.BlockSpec((1, _EMB), lambda i: (0, 0)),
            pl.BlockSpec(memory_space=pltpu.SMEM),
        ],
        out_specs=pl.BlockSpec((_SBLK,), lambda i: (i,)),
        out_shape=jax.ShapeDtypeStruct((_VOCAB_PAD,), jnp.float32),
    )(tT, W, b)


def _pool_body(xf_hbm, scores_hbm, out_hbm, idx_v, vals0, vals1, res_v,
               semi, sem0, sem1):
    wid = lax.axis_index("s") * _NC + lax.axis_index("c")
    base_sample = wid * _SPW

    pltpu.async_copy(
        xf_hbm.at[pl.ds(base_sample * _L, _SPW * _L)], idx_v, semi
    ).wait()

    vals = (vals0, vals1)
    sems = (sem0, sem1)

    def gather_start(c, buf):
        pltpu.async_copy(
            scores_hbm.at[idx_v.at[pl.ds(c * _VCHUNK, _VCHUNK)]], vals[buf],
            sems[buf])

    def gather_wait(c, buf):
        pltpu.make_async_copy(
            scores_hbm.at[idx_v.at[pl.ds(c * _VCHUNK, _VCHUNK)]], vals[buf],
            sems[buf]).wait()

    gather_start(0, 0)
    gather_start(1, 1)

    def pair_body(i, carry):
        c0 = i * 2
        for sub in range(2):
            c = c0 + sub
            gather_wait(c, sub)
            for h in range(_GPC):
                accs = [jnp.zeros((16,), jnp.float32) for _ in range(4)]
                base = h * _CHUNK
                for j in range(_L):
                    v = vals[sub][pl.ds(base + j * _GRP, _GRP)]
                    accs[j % 4] = accs[j % 4] + v
                tot = (accs[0] + accs[1]) + (accs[2] + accs[3])
                z = tot * (1.0 / _L)
                g = c * _GPC + h
                res_v[pl.ds(g * _GRP, _GRP)] = 1.0 / (1.0 + jnp.exp(-z))

            @pl.when(c + 2 < _NCHK)
            def _():
                gather_start(c + 2, sub)
        return carry

    lax.fori_loop(0, _NCHK // 2, pair_body, 0)
    pltpu.sync_copy(res_v, out_hbm.at[pl.ds(base_sample, _SPW)])


def _pool(x_flat, scores):
    mesh = plsc.VectorSubcoreMesh(core_axis_name="c", subcore_axis_name="s")
    return pl.kernel(
        _pool_body,
        out_type=jax.ShapeDtypeStruct((_B,), jnp.float32),
        mesh=mesh,
        compiler_params=pltpu.CompilerParams(needs_layout_passes=False),
        scratch_types=[
            pltpu.VMEM((_SPW * _L,), jnp.int32),
            pltpu.VMEM((_VCHUNK,), jnp.float32),
            pltpu.VMEM((_VCHUNK,), jnp.float32),
            pltpu.VMEM((_SPW,), jnp.float32),
            pltpu.SemaphoreType.DMA,
            pltpu.SemaphoreType.DMA,
            pltpu.SemaphoreType.DMA,
        ],
    )(x_flat, scores)


def kernel(x, table, W, b):
    scores = _vocab_scores(table, W, b)
    x_t = x.reshape(_B // _GRP, _GRP, _L).swapaxes(1, 2).reshape(_B * _L)
    p = _pool(x_t, scores)
    return jnp.round(p.reshape(_B, 1), decimals=4)


# speedup vs baseline: 2.0322x; 2.0322x over previous
import functools

import jax
import jax.numpy as jnp
from jax import lax
from jax.experimental import pallas as pl
from jax.experimental.pallas import tpu as pltpu
from jax.experimental.pallas import tpu_sc as plsc

_VOCAB = 1000000
_EMB = 16
_B = 16384
_L = 200

_NC = 2
_NS = 16
_NW = _NC * _NS
_SPW = _B // _NW
_GRP = 16
_NGRP = _SPW // _GRP
_CHUNK = _GRP * _L
_GPC = 2
_VCHUNK = _CHUNK * _GPC
_NCHK = _NGRP // _GPC


def _scores_body(t_ref, w_ref, b_ref, o_ref):
    d = lax.dot_general(
        w_ref[:], t_ref[:],
        dimension_numbers=(((1,), (0,)), ((), ())),
        preferred_element_type=jnp.float32,
    )
    o_ref[:] = d.reshape(o_ref.shape) + b_ref[0]


_SBLK = 32768
_VOCAB_PAD = 1015808


def _vocab_scores(table, W, b):
    tT = table.T
    return pl.pallas_call(
        _scores_body,
        grid=(_VOCAB_PAD // _SBLK,),
        in_specs=[
            pl.BlockSpec((_EMB, _SBLK), lambda i: (0, i)),
            pl.BlockSpec((1, _EMB), lambda i: (0, 0)),
            pl.BlockSpec(memory_space=pltpu.SMEM),
        ],
        out_specs=pl.BlockSpec((_SBLK,), lambda i: (i,)),
        out_shape=jax.ShapeDtypeStruct((_VOCAB_PAD,), jnp.float32),
    )(tT, W, b)


def _pool_body(xf_hbm, scores_hbm, out_hbm, idx0, idx1, vals0, vals1, res_v,
               scores_spm, semi0, semi1, sem0, sem1):
    wid = lax.axis_index("s") * _NC + lax.axis_index("c")
    sid = lax.axis_index("s")
    base_sample = wid * _SPW
    lane = lax.iota(jnp.int32, 16)
    gidx0 = lane * _L

    @pl.when(sid == 0)
    def _():
        pltpu.sync_copy(scores_hbm, scores_spm)

    idxs = (idx0, idx1)
    isems = (semi0, semi1)
    vals = (vals0, vals1)
    sems = (sem0, sem1)

    def idx_src(c):
        return xf_hbm.at[pl.ds((base_sample * _L) + c * _VCHUNK, _VCHUNK)]

    def idx_start(c, buf):
        pltpu.async_copy(idx_src(c), idxs[buf], isems[buf])

    def idx_wait(c, buf):
        pltpu.make_async_copy(idx_src(c), idxs[buf], isems[buf]).wait()

    def gather_start(buf):
        pltpu.async_copy(scores_spm.at[idxs[buf]], vals[buf], sems[buf])

    def gather_wait(buf):
        pltpu.make_async_copy(scores_spm.at[idxs[buf]], vals[buf],
                              sems[buf]).wait()

    idx_start(0, 0)
    idx_start(1, 1)
    plsc.subcore_barrier()
    idx_wait(0, 0)
    gather_start(0)

    def pair_body(i, carry):
        c0 = i * 2
        for sub in range(2):
            c = c0 + sub

            @pl.when(c + 1 < _NCHK)
            def _():
                idx_wait(c + 1, 1 - sub)
                gather_start(1 - sub)

            gather_wait(sub)

            @pl.when(c + 2 < _NCHK)
            def _():
                idx_start(c + 2, sub)

            for h in range(_GPC):
                accs = [jnp.zeros((16,), jnp.float32) for _ in range(4)]
                base = h * _CHUNK
                for j in range(_L):
                    v = plsc.load_gather(vals[sub], [gidx0 + (base + j)])
                    accs[j % 4] = accs[j % 4] + v
                tot = (accs[0] + accs[1]) + (accs[2] + accs[3])
                z = tot * (1.0 / _L)
                g = c * _GPC + h
                res_v[pl.ds(g * _GRP, _GRP)] = 1.0 / (1.0 + jnp.exp(-z))
        return carry

    lax.fori_loop(0, _NCHK // 2, pair_body, 0)
    pltpu.sync_copy(res_v, out_hbm.at[pl.ds(base_sample, _SPW)])


def _pool(x_flat, scores):
    mesh = plsc.VectorSubcoreMesh(core_axis_name="c", subcore_axis_name="s")
    return pl.kernel(
        _pool_body,
        out_type=jax.ShapeDtypeStruct((_B,), jnp.float32),
        mesh=mesh,
        compiler_params=pltpu.CompilerParams(needs_layout_passes=False),
        scratch_types=[
            pltpu.VMEM((_VCHUNK,), jnp.int32),
            pltpu.VMEM((_VCHUNK,), jnp.int32),
            pltpu.VMEM((_VCHUNK,), jnp.float32),
            pltpu.VMEM((_VCHUNK,), jnp.float32),
            pltpu.VMEM((_SPW,), jnp.float32),
            pltpu.VMEM_SHARED((_VOCAB_PAD,), jnp.float32),
            pltpu.SemaphoreType.DMA,
            pltpu.SemaphoreType.DMA,
            pltpu.SemaphoreType.DMA,
            pltpu.SemaphoreType.DMA,
        ],
    )(x_flat, scores)


def kernel(x, table, W, b):
    scores = _vocab_scores(table, W, b)
    p = _pool(x.reshape(_B * _L), scores)
    return jnp.round(p.reshape(_B, 1), decimals=4)


# speedup vs baseline: 2.1258x; 1.0460x over previous
import functools

import jax
import jax.numpy as jnp
from jax import lax
from jax.experimental import pallas as pl
from jax.experimental.pallas import tpu as pltpu
from jax.experimental.pallas import tpu_sc as plsc

_VOCAB = 1000000
_EMB = 16
_B = 16384
_L = 200

_NC = 2
_NS = 16
_NW = _NC * _NS
_SPW = _B // _NW
_GRP = 16
_NGRP = _SPW // _GRP
_CHUNK = _GRP * _L
_GPC = 2
_VCHUNK = _CHUNK * _GPC
_NCHK = _NGRP // _GPC


def _scores_body(t_ref, w_ref, b_ref, o_ref):
    d = lax.dot_general(
        w_ref[:], t_ref[:],
        dimension_numbers=(((1,), (0,)), ((), ())),
        preferred_element_type=jnp.float32,
    )
    o_ref[:] = d.reshape(o_ref.shape) + b_ref[0]


_SBLK = 65536
_VOCAB_PAD = 1048576


def _vocab_scores(table, W, b):
    tT = table.T
    return pl.pallas_call(
        _scores_body,
        grid=(_VOCAB_PAD // _SBLK,),
        in_specs=[
            pl.BlockSpec((_EMB, _SBLK), lambda i: (0, i)),
            pl.BlockSpec((1, _EMB), lambda i: (0, 0)),
            pl.BlockSpec(memory_space=pltpu.SMEM),
        ],
        out_specs=pl.BlockSpec((_SBLK,), lambda i: (i,)),
        out_shape=jax.ShapeDtypeStruct((_VOCAB_PAD,), jnp.float32),
    )(tT, W, b)


def _pool_body(xf_hbm, scores_hbm, out_hbm, idx0, idx1, vals0, vals1, res_v,
               scores_spm, semi0, semi1, sem0, sem1):
    wid = lax.axis_index("s") * _NC + lax.axis_index("c")
    sid = lax.axis_index("s")
    base_sample = wid * _SPW
    lane = lax.iota(jnp.int32, 16)
    gidx0 = lane * _L

    @pl.when(sid == 0)
    def _():
        pltpu.sync_copy(scores_hbm, scores_spm)

    idxs = (idx0, idx1)
    isems = (semi0, semi1)
    vals = (vals0, vals1)
    sems = (sem0, sem1)

    def idx_src(c):
        return xf_hbm.at[pl.ds((base_sample * _L) + c * _VCHUNK, _VCHUNK)]

    def idx_start(c, buf):
        pltpu.async_copy(idx_src(c), idxs[buf], isems[buf])

    def idx_wait(c, buf):
        pltpu.make_async_copy(idx_src(c), idxs[buf], isems[buf]).wait()

    def gather_start(buf):
        pltpu.async_copy(scores_spm.at[idxs[buf]], vals[buf], sems[buf])

    def gather_wait(buf):
        pltpu.make_async_copy(scores_spm.at[idxs[buf]], vals[buf],
                              sems[buf]).wait()

    idx_start(0, 0)
    idx_start(1, 1)
    plsc.subcore_barrier()
    idx_wait(0, 0)
    gather_start(0)

    def pair_body(i, carry):
        c0 = i * 2
        for sub in range(2):
            c = c0 + sub

            @pl.when(c + 1 < _NCHK)
            def _():
                idx_wait(c + 1, 1 - sub)
                gather_start(1 - sub)

            gather_wait(sub)

            @pl.when(c + 2 < _NCHK)
            def _():
                idx_start(c + 2, sub)

            for h in range(_GPC):
                accs = [jnp.zeros((16,), jnp.float32) for _ in range(4)]
                base = h * _CHUNK
                for j in range(_L):
                    v = plsc.load_gather(vals[sub], [gidx0 + (base + j)])
                    accs[j % 4] = accs[j % 4] + v
                tot = (accs[0] + accs[1]) + (accs[2] + accs[3])
                z = tot * (1.0 / _L)
                g = c * _GPC + h
                res_v[pl.ds(g * _GRP, _GRP)] = 1.0 / (1.0 + jnp.exp(-z))
        return carry

    lax.fori_loop(0, _NCHK // 2, pair_body, 0)
    pltpu.sync_copy(res_v, out_hbm.at[pl.ds(base_sample, _SPW)])


def _pool(x_flat, scores):
    mesh = plsc.VectorSubcoreMesh(core_axis_name="c", subcore_axis_name="s")
    return pl.kernel(
        _pool_body,
        out_type=jax.ShapeDtypeStruct((_B,), jnp.float32),
        mesh=mesh,
        compiler_params=pltpu.CompilerParams(needs_layout_passes=False),
        scratch_types=[
            pltpu.VMEM((_VCHUNK,), jnp.int32),
            pltpu.VMEM((_VCHUNK,), jnp.int32),
            pltpu.VMEM((_VCHUNK,), jnp.float32),
            pltpu.VMEM((_VCHUNK,), jnp.float32),
            pltpu.VMEM((_SPW,), jnp.float32),
            pltpu.VMEM_SHARED((_VOCAB_PAD,), jnp.float32),
            pltpu.SemaphoreType.DMA,
            pltpu.SemaphoreType.DMA,
            pltpu.SemaphoreType.DMA,
            pltpu.SemaphoreType.DMA,
        ],
    )(x_flat, scores)


def kernel(x, table, W, b):
    scores = _vocab_scores(table, W, b)
    p = _pool(x.reshape(_B * _L), scores)
    return jnp.round(p.reshape(_B, 1), decimals=4)


# speedup vs baseline: 2.1292x; 1.0016x over previous
import jax
import jax.numpy as jnp
from jax import lax
from jax.experimental import pallas as pl
from jax.experimental.pallas import tpu as pltpu
from jax.experimental.pallas import tpu_sc as plsc

_VOCAB = 1000000
_EMB = 16
_B = 16384
_L = 200

_NC = 2
_NS = 16
_NW = _NC * _NS
_SPW = _B // _NW
_GRP = 16
_NGRP = _SPW // _GRP
_CHUNK = _GRP * _L
_GPC = 2
_VCHUNK = _CHUNK * _GPC
_NCHK = _NGRP // _GPC


def _scores_body(t_ref, w_ref, b_ref, o_ref):
    d = lax.dot_general(
        w_ref[:], t_ref[:],
        dimension_numbers=(((1,), (0,)), ((), ())),
        preferred_element_type=jnp.float32,
    )
    o_ref[:] = d.reshape(o_ref.shape) + b_ref[0]


_SBLK = 65536
_VOCAB_PAD = 1048576


def _vocab_scores(table, W, b):
    tT = table.T
    return pl.pallas_call(
        _scores_body,
        grid=(_VOCAB_PAD // _SBLK,),
        in_specs=[
            pl.BlockSpec((_EMB, _SBLK), lambda i: (0, i)),
            pl.BlockSpec((1, _EMB), lambda i: (0, 0)),
            pl.BlockSpec(memory_space=pltpu.SMEM),
        ],
        out_specs=pl.BlockSpec((_SBLK,), lambda i: (i,)),
        out_shape=jax.ShapeDtypeStruct((_VOCAB_PAD,), jnp.float32),
    )(tT, W, b)


def _pool_body(xf_hbm, scores_hbm, out_hbm, idx0, idx1, vals0, vals1, res_v,
               scores_spm, semi0, semi1, sem0, sem1):
    wid = lax.axis_index("s") * _NC + lax.axis_index("c")
    sid = lax.axis_index("s")
    base_sample = wid * _SPW
    lane = lax.iota(jnp.int32, 16)
    gidx0 = lane * _L

    @pl.when(sid == 0)
    def _():
        pltpu.sync_copy(scores_hbm, scores_spm)

    idxs = (idx0, idx1)
    isems = (semi0, semi1)
    vals = (vals0, vals1)
    sems = (sem0, sem1)

    def idx_src(c):
        return xf_hbm.at[pl.ds((base_sample * _L) + c * _VCHUNK, _VCHUNK)]

    def idx_start(c, buf):
        pltpu.async_copy(idx_src(c), idxs[buf], isems[buf])

    def idx_wait(c, buf):
        pltpu.make_async_copy(idx_src(c), idxs[buf], isems[buf]).wait()

    def gather_start(buf):
        pltpu.async_copy(scores_spm.at[idxs[buf]], vals[buf], sems[buf])

    def gather_wait(buf):
        pltpu.make_async_copy(scores_spm.at[idxs[buf]], vals[buf],
                              sems[buf]).wait()

    idx_start(0, 0)
    idx_start(1, 1)
    plsc.subcore_barrier()
    idx_wait(0, 0)
    gather_start(0)

    def pair_body(i, carry):
        c0 = i * 2
        for sub in range(2):
            c = c0 + sub

            @pl.when(c + 1 < _NCHK)
            def _():
                idx_wait(c + 1, 1 - sub)
                gather_start(1 - sub)

            gather_wait(sub)

            @pl.when(c + 2 < _NCHK)
            def _():
                idx_start(c + 2, sub)

            for h in range(_GPC):
                accs = [jnp.zeros((16,), jnp.float32) for _ in range(4)]
                base = h * _CHUNK
                for j in range(_L):
                    v = plsc.load_gather(vals[sub], [gidx0 + (base + j)])
                    accs[j % 4] = accs[j % 4] + v
                tot = (accs[0] + accs[1]) + (accs[2] + accs[3])
                z = tot * (1.0 / _L)
                g = c * _GPC + h
                res_v[pl.ds(g * _GRP, _GRP)] = 1.0 / (1.0 + jnp.exp(-z))
        return carry

    lax.fori_loop(0, _NCHK // 2, pair_body, 0)
    pltpu.sync_copy(res_v, out_hbm.at[pl.ds(base_sample, _SPW)])


def _pool(x_flat, scores):
    mesh = plsc.VectorSubcoreMesh(core_axis_name="c", subcore_axis_name="s")
    return pl.kernel(
        _pool_body,
        out_type=jax.ShapeDtypeStruct((_B,), jnp.float32),
        mesh=mesh,
        compiler_params=pltpu.CompilerParams(needs_layout_passes=False),
        scratch_types=[
            pltpu.VMEM((_VCHUNK,), jnp.int32),
            pltpu.VMEM((_VCHUNK,), jnp.int32),
            pltpu.VMEM((_VCHUNK,), jnp.float32),
            pltpu.VMEM((_VCHUNK,), jnp.float32),
            pltpu.VMEM((_SPW,), jnp.float32),
            pltpu.VMEM_SHARED((_VOCAB_PAD,), jnp.float32),
            pltpu.SemaphoreType.DMA,
            pltpu.SemaphoreType.DMA,
            pltpu.SemaphoreType.DMA,
            pltpu.SemaphoreType.DMA,
        ],
    )(x_flat, scores)


def kernel(x, table, W, b):
    scores = _vocab_scores(table, W, b)
    p = _pool(x.reshape(_B * _L), scores)
    return jnp.round(p.reshape(_B, 1), decimals=4)
